# baseline (speedup 1.0000x reference)
import functools
import jax
import jax.numpy as jnp
from jax import lax
from jax.experimental import pallas as pl
from jax.experimental.pallas import tpu as pltpu
from jax.experimental.pallas import tpu_sc as plsc

_K = 10
_B = 128
_V = 32768
_LANES = 128
_VI = _V // 2
_CW = _LANES
_NCH = _VI // _CW
_CBLK = 4096
_NC = _VI // _CBLK
_NSL = _CBLK // _LANES
_NEG = -3.0e38

_SC_NC = 2
_SC_NS = 16
_NW = _SC_NC * _SC_NS
_NIDX = _B * _K
_BPW = _NIDX // _NW


def _keyed_chunk_max(x_ref, m_ref, step):
    cols = []
    for j in range(_NSL):
        v = x_ref[:, j * _LANES:(j + 1) * _LANES]
        lo = v & 0xFFFF
        hi = (v >> 16) & 0xFFFF
        klo = jnp.where(lo >= 0x8000, 0xFFFF - lo, lo + 0x8000)
        khi = jnp.where(hi >= 0x8000, 0xFFFF - hi, hi + 0x8000)
        m = jnp.maximum(klo, khi)
        cols.append(jnp.max(m, axis=1, keepdims=True))
    m_ref[step] = jnp.concatenate(cols, axis=1)


def _topk_chunk_idx(m_ref):
    cand = jnp.concatenate([m_ref[c] for c in range(_NC)], axis=1)
    idx = lax.broadcasted_iota(jnp.int32, cand.shape, 1)
    outs = []
    for _ in range(_K):
        m = jnp.max(cand, axis=1, keepdims=True)
        eq = cand == m
        pos = jnp.min(jnp.where(eq, idx, _NCH), axis=1, keepdims=True)
        cand = jnp.where(idx == pos, -1, cand)
        outs.append(pos)
    rows = lax.broadcasted_iota(jnp.int32, (_B, _K), 0)
    return jnp.concatenate(outs, axis=1) + rows * _NCH


def _select_body(llm_ref, slm_ref, il_ref, is_ref, ml_ref, ms_ref):
    step = pl.program_id(0)
    _keyed_chunk_max(llm_ref, ml_ref, step)
    _keyed_chunk_max(slm_ref, ms_ref, step)

    @pl.when(step == _NC - 1)
    def _():
        il_ref[...] = _topk_chunk_idx(ml_ref)
        is_ref[...] = _topk_chunk_idx(ms_ref)


def _select_chunks(iv_l, iv_s):
    return pl.pallas_call(
        _select_body,
        grid=(_NC,),
        in_specs=[
            pl.BlockSpec((_B, _CBLK), lambda c: (0, c)),
            pl.BlockSpec((_B, _CBLK), lambda c: (0, c)),
        ],
        out_specs=[
            pl.BlockSpec((_B, _K), lambda c: (0, 0)),
            pl.BlockSpec((_B, _K), lambda c: (0, 0)),
        ],
        out_shape=[
            jax.ShapeDtypeStruct((_B, _K), jnp.int32),
            jax.ShapeDtypeStruct((_B, _K), jnp.int32),
        ],
        scratch_shapes=[
            pltpu.VMEM((_NC, _B, _NSL), jnp.int32),
            pltpu.VMEM((_NC, _B, _NSL), jnp.int32),
        ],
    )(iv_l, iv_s)


@functools.lru_cache(maxsize=None)
def _build_sc_gather():
    mesh = plsc.VectorSubcoreMesh(core_axis_name="c", subcore_axis_name="s")

    @functools.partial(
        pl.kernel,
        mesh=mesh,
        out_type=[
            jax.ShapeDtypeStruct((_NIDX, _CW), jnp.int32),
            jax.ShapeDtypeStruct((_NIDX, _CW), jnp.int32),
        ],
        scratch_types=[
            pltpu.VMEM((_BPW,), jnp.int32),
            pltpu.VMEM((_BPW, _CW), jnp.int32),
            pltpu.SemaphoreType.DMA,
        ],
    )
    def gather(tl_hbm, ts_hbm, il_hbm, is_hbm, ol_hbm, os_hbm,
               idx_v, rows_v, sem):
        wid = lax.axis_index("s") * _SC_NC + lax.axis_index("c")
        base = wid * _BPW
        pltpu.sync_copy(il_hbm.at[pl.ds(base, _BPW)], idx_v)
        pltpu.async_copy(tl_hbm.at[idx_v], rows_v, sem).wait()
        pltpu.sync_copy(rows_v, ol_hbm.at[pl.ds(base, _BPW)])
        pltpu.sync_copy(is_hbm.at[pl.ds(base, _BPW)], idx_v)
        pltpu.async_copy(ts_hbm.at[idx_v], rows_v, sem).wait()
        pltpu.sync_copy(rows_v, os_hbm.at[pl.ds(base, _BPW)])

    return gather


def _sc_gather(tl, ts, il, isx):
    return _build_sc_gather()(tl, ts, il, isx)


def _merge_topk(cand):
    idx = lax.broadcasted_iota(jnp.int32, cand.shape, 1)
    n = cand.shape[1]
    outs = []
    for _ in range(_K):
        m = jnp.max(cand, axis=1, keepdims=True)
        eq = cand == m
        pos = jnp.min(jnp.where(eq, idx, n), axis=1, keepdims=True)
        cand = jnp.where(idx == pos, _NEG, cand)
        outs.append(m)
    return jnp.concatenate(outs, axis=1)


def _finish_body(cl_ref, cs_ref, w1t_ref, b1_ref, w2t_ref, b2_ref,
                 w3t_ref, b3_ref, out_ref):
    tl = _merge_topk(cl_ref[...])
    ts = _merge_topk(cs_ref[...])
    c = jnp.concatenate([tl, ts], axis=1)
    z1 = jnp.dot(c, w1t_ref[...], preferred_element_type=jnp.float32) + b1_ref[...]
    h1 = jnp.maximum(z1, 0.0)
    z2 = jnp.dot(h1, w2t_ref[...], preferred_element_type=jnp.float32) + b2_ref[...]
    h2 = jnp.maximum(z2, 0.0)
    z3 = jnp.dot(h2, w3t_ref[...], preferred_element_type=jnp.float32) + b3_ref[...]
    raw = jax.nn.sigmoid(z3)
    out_ref[...] = raw / jnp.sum(raw, axis=1, keepdims=True)


def _finish(cl32, cs32, w1t, b1r, w2t, b2r, w3t, b3r):
    full = lambda shape: pl.BlockSpec(shape, lambda: (0,) * len(shape))
    return pl.pallas_call(
        _finish_body,
        in_specs=[full(cl32.shape), full(cs32.shape),
                  full(w1t.shape), full(b1r.shape),
                  full(w2t.shape), full(b2r.shape),
                  full(w3t.shape), full(b3r.shape)],
        out_specs=full((_B, 2)),
        out_shape=jax.ShapeDtypeStruct((_B, 2), jnp.float32),
    )(cl32, cs32, w1t, b1r, w2t, b2r, w3t, b3r)


def _cand_values(g):
    h = jax.lax.bitcast_convert_type(g, jnp.float16)
    return h.reshape(_B, _K * _CW * 2).astype(jnp.float32)


def kernel(llm_logits, slm_logits, W1, b1, W2, b2, W3, b3):
    iv_l = jax.lax.bitcast_convert_type(
        llm_logits.reshape(_B, _VI, 2), jnp.int32)
    iv_s = jax.lax.bitcast_convert_type(
        slm_logits.reshape(_B, _VI, 2), jnp.int32)

    idx_l, idx_s = _select_chunks(iv_l, iv_s)

    gl, gs = _sc_gather(iv_l.reshape(_B * _NCH, _CW),
                        iv_s.reshape(_B * _NCH, _CW),
                        idx_l.reshape(_NIDX), idx_s.reshape(_NIDX))

    w1t = W1.T.astype(jnp.float32)
    w2t = W2.T.astype(jnp.float32)
    w3t = W3.T.astype(jnp.float32)
    b1r = b1.reshape(1, -1).astype(jnp.float32)
    b2r = b2.reshape(1, -1).astype(jnp.float32)
    b3r = b3.reshape(1, -1).astype(jnp.float32)

    out = _finish(_cand_values(gl), _cand_values(gs),
                  w1t, b1r, w2t, b2r, w3t, b3r)
    return out.astype(jnp.float16)


# speedup vs baseline: 7.5005x; 7.5005x over previous
import jax
import jax.numpy as jnp
from jax.experimental import pallas as pl
from jax.experimental.pallas import tpu as pltpu

_K = 10
_B = 128
_V = 32768
_RG = 16
_LANES = 128
_CHUNK = 8192
_NC = _V // _CHUNK
_NSL = _CHUNK // _LANES
_NG = _B // _RG
_NEG = -3.0e38


def _insert_chunk(x_ref, tiles_ref):
    tiles = [tiles_ref[:, k * _LANES:(k + 1) * _LANES] for k in range(_K)]
    for j in range(_NSL):
        v = x_ref[:, j * _LANES:(j + 1) * _LANES]
        for k in range(_K):
            t = tiles[k]
            hi = jnp.maximum(t, v)
            v = jnp.minimum(t, v)
            tiles[k] = hi
    for k in range(_K):
        tiles_ref[:, k * _LANES:(k + 1) * _LANES] = tiles[k]


def _merge_topk(tiles_ref):
    cand = tiles_ref[...]
    idx = jax.lax.broadcasted_iota(jnp.int32, cand.shape, 1)
    outs = []
    for _ in range(_K):
        m = jnp.max(cand, axis=1, keepdims=True)
        eq = cand == m
        pos = jnp.min(jnp.where(eq, idx, _K * _LANES), axis=1, keepdims=True)
        cand = jnp.where(idx == pos, _NEG, cand)
        outs.append(m)
    return jnp.concatenate(outs, axis=1)


def _mlp(c, w1t_ref, b1_ref, w2t_ref, b2_ref, w3t_ref, b3_ref):
    z1 = jnp.dot(c, w1t_ref[...], preferred_element_type=jnp.float32) + b1_ref[...]
    h1 = jnp.maximum(z1, 0.0)
    z2 = jnp.dot(h1, w2t_ref[...], preferred_element_type=jnp.float32) + b2_ref[...]
    h2 = jnp.maximum(z2, 0.0)
    z3 = jnp.dot(h2, w3t_ref[...], preferred_element_type=jnp.float32) + b3_ref[...]
    raw = jax.nn.sigmoid(z3)
    return raw / jnp.sum(raw, axis=1, keepdims=True)


def _kernel_body(llm_ref, slm_ref, w1t_ref, b1_ref, w2t_ref, b2_ref,
                 w3t_ref, b3_ref, out_ref, tl_ref, ts_ref, til_ref, tis_ref):
    g = pl.program_id(0)
    c = pl.program_id(1)

    @pl.when(c == 0)
    def _():
        neg = jnp.full((_RG, _K * _LANES), _NEG, dtype=jnp.float32)
        til_ref[...] = neg
        tis_ref[...] = neg

    _insert_chunk(llm_ref, til_ref)
    _insert_chunk(slm_ref, tis_ref)

    @pl.when(c == _NC - 1)
    def _():
        tl_ref[pl.ds(g * _RG, _RG), :] = _merge_topk(til_ref)
        ts_ref[pl.ds(g * _RG, _RG), :] = _merge_topk(tis_ref)

    @pl.when((g == _NG - 1) & (c == _NC - 1))
    def _():
        cc = jnp.concatenate([tl_ref[...], ts_ref[...]], axis=1)
        out_ref[...] = _mlp(cc, w1t_ref, b1_ref, w2t_ref, b2_ref,
                            w3t_ref, b3_ref)


def kernel(llm_logits, slm_logits, W1, b1, W2, b2, W3, b3):
    llm32 = llm_logits.astype(jnp.float32)
    slm32 = slm_logits.astype(jnp.float32)
    w1t = W1.T.astype(jnp.float32)
    w2t = W2.T.astype(jnp.float32)
    w3t = W3.T.astype(jnp.float32)
    b1r = b1.reshape(1, -1).astype(jnp.float32)
    b2r = b2.reshape(1, -1).astype(jnp.float32)
    b3r = b3.reshape(1, -1).astype(jnp.float32)

    full = lambda shape: pl.BlockSpec(shape, lambda g, c: (0,) * len(shape))
    out = pl.pallas_call(
        _kernel_body,
        grid=(_NG, _NC),
        in_specs=[
            pl.BlockSpec((_RG, _CHUNK), lambda g, c: (g, c)),
            pl.BlockSpec((_RG, _CHUNK), lambda g, c: (g, c)),
            full(w1t.shape), full(b1r.shape),
            full(w2t.shape), full(b2r.shape),
            full(w3t.shape), full(b3r.shape),
        ],
        out_specs=pl.BlockSpec((_B, 2), lambda g, c: (0, 0)),
        out_shape=jax.ShapeDtypeStruct((_B, 2), jnp.float32),
        scratch_shapes=[
            pltpu.VMEM((_B, _K), jnp.float32),
            pltpu.VMEM((_B, _K), jnp.float32),
            pltpu.VMEM((_RG, _K * _LANES), jnp.float32),
            pltpu.VMEM((_RG, _K * _LANES), jnp.float32),
        ],
    )(llm32, slm32, w1t, b1r, w2t, b2r, w3t, b3r)
    return out.astype(jnp.float16)
